# Initial kernel scaffold; baseline (speedup 1.0000x reference)
import jax
import jax.numpy as jnp
from jax.experimental import pallas as pl


def kernel(x, adj, W1, b1, W2, b2, W3, b3):
    raise NotImplementedError("write your pallas kernel here")


# speedup vs baseline: 12.7584x; 12.7584x over previous
import functools

import jax
import jax.numpy as jnp
from jax import lax
from jax.experimental import pallas as pl
from jax.experimental.pallas import tpu as pltpu
from jax.experimental.pallas import tpu_sc as plsc

N = 10000
E = 320000
F = 128
NPAD = 10240
CW = 128
CHUNK = 128
NCHUNKS = E // CHUNK
NC, NS = 2, 16
NW = NC * NS
IBLK = 400
NIBLK = N // IBLK
CNT_PER_TILE = NPAD // NS
RBLK = 1000

_mesh = plsc.VectorSubcoreMesh(core_axis_name="c", subcore_axis_name="s",
                               num_cores=NC, num_subcores=NS)


def _worker_id():
    return lax.axis_index("s") * NC + lax.axis_index("c")


def _deg_body(dst_hbm, zeros_hbm, ones_hbm, out_hbm, dbuf, ones_v, acc):
    c = lax.axis_index("c")
    s = lax.axis_index("s")
    wid = _worker_id()
    row0 = s * CNT_PER_TILE
    pltpu.sync_copy(zeros_hbm.at[pl.ds(row0, CNT_PER_TILE)],
                    acc.at[pl.ds(row0, CNT_PER_TILE)])
    pltpu.sync_copy(ones_hbm, ones_v)
    plsc.subcore_barrier()

    def chunk(j, carry):
        cid = j * NW + wid

        @pl.when(cid < NCHUNKS)
        def _():
            pltpu.sync_copy(dst_hbm.at[pl.ds(cid * CHUNK, CHUNK)], dbuf)
            pltpu.sync_copy(ones_v, acc.at[dbuf], add=True)

        return carry

    lax.fori_loop(0, (NCHUNKS + NW - 1) // NW, chunk, 0)
    plsc.subcore_barrier()
    pltpu.sync_copy(acc.at[pl.ds(row0, CNT_PER_TILE)],
                    out_hbm.at[c, pl.ds(row0, CNT_PER_TILE)])


_deg_kernel = functools.partial(
    pl.kernel,
    out_type=jax.ShapeDtypeStruct((NC, NPAD, CW), jnp.float32),
    mesh=_mesh,
    scratch_types=[
        pltpu.VMEM((CHUNK,), jnp.int32),
        pltpu.VMEM((CHUNK, CW), jnp.float32),
        pltpu.VMEM_SHARED((NPAD, CW), jnp.float32),
    ],
)(_deg_body)


def _agg_body(hs_hbm, src_hbm, dst_hbm, out_hbm, sbuf, dbuf, rows, acc, sem):
    c = lax.axis_index("c")
    s = lax.axis_index("s")
    wid = _worker_id()

    def init_blk(j, carry):
        bid = j * NS + s

        @pl.when(bid < NIBLK)
        def _():
            pltpu.sync_copy(hs_hbm.at[pl.ds(bid * IBLK, IBLK)],
                            acc.at[pl.ds(bid * IBLK, IBLK)])

        return carry

    lax.fori_loop(0, (NIBLK + NS - 1) // NS, init_blk, 0)
    plsc.subcore_barrier()

    def chunk(j, carry):
        cid = j * NW + wid

        @pl.when(cid < NCHUNKS)
        def _():
            pltpu.sync_copy(src_hbm.at[pl.ds(cid * CHUNK, CHUNK)], sbuf)
            pltpu.sync_copy(dst_hbm.at[pl.ds(cid * CHUNK, CHUNK)], dbuf)
            pltpu.async_copy(hs_hbm.at[sbuf], rows, sem).wait()
            pltpu.sync_copy(rows, acc.at[dbuf], add=True)

        return carry

    lax.fori_loop(0, (NCHUNKS + NW - 1) // NW, chunk, 0)
    plsc.subcore_barrier()

    def out_blk(j, carry):
        bid = j * NS + s

        @pl.when(bid < NIBLK)
        def _():
            pltpu.sync_copy(acc.at[pl.ds(bid * IBLK, IBLK)],
                            out_hbm.at[c, pl.ds(bid * IBLK, IBLK)])

        return carry

    lax.fori_loop(0, (NIBLK + NS - 1) // NS, out_blk, 0)


_agg_kernel = functools.partial(
    pl.kernel,
    out_type=jax.ShapeDtypeStruct((NC, N, F), jnp.float32),
    mesh=_mesh,
    scratch_types=[
        pltpu.VMEM((CHUNK,), jnp.int32),
        pltpu.VMEM((CHUNK,), jnp.int32),
        pltpu.VMEM((CHUNK, F), jnp.float32),
        pltpu.VMEM_SHARED((N, F), jnp.float32),
        pltpu.SemaphoreType.DMA,
    ],
)(_agg_body)


def _dinv_body(cnt_ref, o_ref):
    deg = cnt_ref[0] + cnt_ref[1] + 1.0
    o_ref[...] = lax.rsqrt(deg)


def _dinv_kernel(cnt):
    return pl.pallas_call(
        _dinv_body,
        out_shape=jax.ShapeDtypeStruct((NPAD // F, F), jnp.float32),
    )(cnt)


def _mm_scale_body(h_ref, w_ref, dinv_ref, o_ref):
    hw = jnp.dot(h_ref[...], w_ref[...], preferred_element_type=jnp.float32)
    o_ref[...] = hw * dinv_ref[...]


def _mm_scale(h, w, dinv_col):
    return pl.pallas_call(
        _mm_scale_body,
        grid=(N // RBLK,),
        in_specs=[
            pl.BlockSpec((RBLK, F), lambda i: (i, 0)),
            pl.BlockSpec((F, F), lambda i: (0, 0)),
            pl.BlockSpec((RBLK, 1), lambda i: (i, 0)),
        ],
        out_specs=pl.BlockSpec((RBLK, F), lambda i: (i, 0)),
        out_shape=jax.ShapeDtypeStruct((N, F), jnp.float32),
    )(h, w, dinv_col)


def _combine_mm_body(pa_ref, pb_ref, hs_ref, dinv_ref, b_ref, w_ref, o_ref):
    a = dinv_ref[...] * (pa_ref[...] + pb_ref[...] - hs_ref[...]) + b_ref[...]
    t = jnp.maximum(a, 0.0)
    hw = jnp.dot(t, w_ref[...], preferred_element_type=jnp.float32)
    o_ref[...] = hw * dinv_ref[...]


def _combine_mm(pa, pb, hs, dinv_col, b_row, w):
    return pl.pallas_call(
        _combine_mm_body,
        grid=(N // RBLK,),
        in_specs=[
            pl.BlockSpec((RBLK, F), lambda i: (i, 0)),
            pl.BlockSpec((RBLK, F), lambda i: (i, 0)),
            pl.BlockSpec((RBLK, F), lambda i: (i, 0)),
            pl.BlockSpec((RBLK, 1), lambda i: (i, 0)),
            pl.BlockSpec((1, F), lambda i: (0, 0)),
            pl.BlockSpec((F, F), lambda i: (0, 0)),
        ],
        out_specs=pl.BlockSpec((RBLK, F), lambda i: (i, 0)),
        out_shape=jax.ShapeDtypeStruct((N, F), jnp.float32),
    )(pa, pb, hs, dinv_col, b_row, w)


def _final_body(pa_ref, pb_ref, hs_ref, dinv_ref, b_ref, o_ref):
    a = dinv_ref[...] * (pa_ref[...] + pb_ref[...] - hs_ref[...]) + b_ref[...]
    m = jnp.max(a, axis=1, keepdims=True)
    lse = jnp.log(jnp.sum(jnp.exp(a - m), axis=1, keepdims=True)) + m
    o_ref[...] = a - lse


def _final(pa, pb, hs, dinv_col, b_row):
    return pl.pallas_call(
        _final_body,
        grid=(N // RBLK,),
        in_specs=[
            pl.BlockSpec((RBLK, F), lambda i: (i, 0)),
            pl.BlockSpec((RBLK, F), lambda i: (i, 0)),
            pl.BlockSpec((RBLK, F), lambda i: (i, 0)),
            pl.BlockSpec((RBLK, 1), lambda i: (i, 0)),
            pl.BlockSpec((1, F), lambda i: (0, 0)),
        ],
        out_specs=pl.BlockSpec((RBLK, F), lambda i: (i, 0)),
        out_shape=jax.ShapeDtypeStruct((N, F), jnp.float32),
    )(pa, pb, hs, dinv_col, b_row)


_DBG_JNP_DEG = False
_DBG_JNP_AGG = False


def _jnp_deg(dst):
    c = jnp.zeros((NPAD,), jnp.float32).at[dst].add(1.0)
    return jnp.broadcast_to(c[None, :, None] * 0.5, (NC, NPAD, CW))


def _jnp_agg(hs, src, dst):
    a = hs.at[dst].add(hs[src], mode="promise_in_bounds")
    return jnp.stack([a, hs])


def kernel(x, adj, W1, b1, W2, b2, W3, b3):
    src = adj[0].astype(jnp.int32)
    dst = adj[1].astype(jnp.int32)

    if _DBG_JNP_DEG:
        cnt = _jnp_deg(dst)
    else:
        cnt = _deg_kernel(dst,
                          jnp.zeros((NPAD, CW), jnp.float32),
                          jnp.ones((CHUNK, CW), jnp.float32))
    cnt2d = cnt[:, :, 0].reshape(NC, NPAD // F, F)
    dinv_col = _dinv_kernel(cnt2d).reshape(NPAD, 1)[:N]

    agg = _jnp_agg if _DBG_JNP_AGG else (lambda h, s_, d_: _agg_kernel(h, s_, d_))

    hs = _mm_scale(x, W1, dinv_col)
    p = agg(hs, src, dst)
    hs = _combine_mm(p[0], p[1], hs, dinv_col, b1.reshape(1, F), W2)
    p = agg(hs, src, dst)
    hs = _combine_mm(p[0], p[1], hs, dinv_col, b2.reshape(1, F), W3)
    p = agg(hs, src, dst)
    return _final(p[0], p[1], hs, dinv_col, b3.reshape(1, F))


# speedup vs baseline: 19.3223x; 1.5145x over previous
import functools

import jax
import jax.numpy as jnp
from jax import lax
from jax.experimental import pallas as pl
from jax.experimental.pallas import tpu as pltpu
from jax.experimental.pallas import tpu_sc as plsc

N = 10000
E = 320000
F = 128
NPAD = 10240
CW = 128
CHUNK = 128
NCHUNKS = E // CHUNK
NC, NS = 2, 16
NW = NC * NS
IBLK = 400
NIBLK = N // IBLK
CNT_PER_TILE = NPAD // NS
RBLK = 1000

_mesh = plsc.VectorSubcoreMesh(core_axis_name="c", subcore_axis_name="s",
                               num_cores=NC, num_subcores=NS)


def _worker_id():
    return lax.axis_index("s") * NC + lax.axis_index("c")


def _deg_body(dst_hbm, zeros_hbm, ones_hbm, out_hbm, dbuf, ones_v, acc):
    c = lax.axis_index("c")
    s = lax.axis_index("s")
    wid = _worker_id()
    row0 = s * CNT_PER_TILE
    pltpu.sync_copy(zeros_hbm.at[pl.ds(row0, CNT_PER_TILE)],
                    acc.at[pl.ds(row0, CNT_PER_TILE)])
    pltpu.sync_copy(ones_hbm, ones_v)
    plsc.subcore_barrier()

    def chunk(j, carry):
        cid = j * NW + wid

        @pl.when(cid < NCHUNKS)
        def _():
            pltpu.sync_copy(dst_hbm.at[pl.ds(cid * CHUNK, CHUNK)], dbuf)
            pltpu.sync_copy(ones_v, acc.at[dbuf], add=True)

        return carry

    lax.fori_loop(0, (NCHUNKS + NW - 1) // NW, chunk, 0)
    plsc.subcore_barrier()
    pltpu.sync_copy(acc.at[pl.ds(row0, CNT_PER_TILE)],
                    out_hbm.at[c, pl.ds(row0, CNT_PER_TILE)])


_deg_kernel = functools.partial(
    pl.kernel,
    out_type=jax.ShapeDtypeStruct((NC, NPAD, CW), jnp.float32),
    mesh=_mesh,
    scratch_types=[
        pltpu.VMEM((CHUNK,), jnp.int32),
        pltpu.VMEM((CHUNK, CW), jnp.float32),
        pltpu.VMEM_SHARED((NPAD, CW), jnp.float32),
    ],
)(_deg_body)


NBUF = 3
NITER = 81


def _agg_body(hs_hbm, src_hbm, dst_hbm, out_hbm,
              sbufs, dbufs, rows, acc, gsems, ssems):
    c = lax.axis_index("c")
    s = lax.axis_index("s")
    wid = _worker_id()

    def init_blk(j, carry):
        bid = j * NS + s

        @pl.when(bid < NIBLK)
        def _():
            pltpu.sync_copy(hs_hbm.at[pl.ds(bid * IBLK, IBLK)],
                            acc.at[pl.ds(bid * IBLK, IBLK)])

        return carry

    lax.fori_loop(0, (NIBLK + NS - 1) // NS, init_blk, 0)
    plsc.subcore_barrier()

    def load_and_gather(b, j):
        cid = j * NW + wid
        pltpu.sync_copy(src_hbm.at[pl.ds(cid * CHUNK, CHUNK)], sbufs[b])
        pltpu.sync_copy(dst_hbm.at[pl.ds(cid * CHUNK, CHUNK)], dbufs[b])
        pltpu.async_copy(hs_hbm.at[sbufs[b]], rows[b], gsems[b])

    for b in range(NBUF):
        load_and_gather(b, b)

    def cycle(g, carry):
        for b in range(NBUF):
            j = g * NBUF + b
            cid = j * NW + wid

            @pl.when(cid < NCHUNKS)
            def _(b=b):
                pltpu.make_async_copy(hs_hbm.at[sbufs[b]], rows[b],
                                      gsems[b]).wait()
                pltpu.async_copy(rows[b], acc.at[dbufs[b]], ssems[b], add=True)

        for b in range(NBUF):
            j = g * NBUF + b
            cid = j * NW + wid
            cidn = (j + NBUF) * NW + wid

            @pl.when(cid < NCHUNKS)
            def _(b=b):
                pltpu.make_async_copy(rows[b], acc.at[dbufs[b]],
                                      ssems[b]).wait()

                @pl.when(cidn < NCHUNKS)
                def _():
                    load_and_gather(b, j + NBUF)

        return carry

    lax.fori_loop(0, NITER // NBUF, cycle, 0)
    plsc.subcore_barrier()

    def out_blk(j, carry):
        bid = j * NS + s

        @pl.when(bid < NIBLK)
        def _():
            pltpu.sync_copy(acc.at[pl.ds(bid * IBLK, IBLK)],
                            out_hbm.at[c, pl.ds(bid * IBLK, IBLK)])

        return carry

    lax.fori_loop(0, (NIBLK + NS - 1) // NS, out_blk, 0)


_agg_kernel = functools.partial(
    pl.kernel,
    out_type=jax.ShapeDtypeStruct((NC, N, F), jnp.float32),
    mesh=_mesh,
    scratch_types=[
        [pltpu.VMEM((CHUNK,), jnp.int32) for _ in range(NBUF)],
        [pltpu.VMEM((CHUNK,), jnp.int32) for _ in range(NBUF)],
        [pltpu.VMEM((CHUNK, F), jnp.float32) for _ in range(NBUF)],
        pltpu.VMEM_SHARED((N, F), jnp.float32),
        [pltpu.SemaphoreType.DMA for _ in range(NBUF)],
        [pltpu.SemaphoreType.DMA for _ in range(NBUF)],
    ],
)(_agg_body)


def _dinv_body(cnt_ref, o_ref):
    deg = cnt_ref[0] + cnt_ref[1] + 1.0
    o_ref[...] = lax.rsqrt(deg)


def _dinv_kernel(cnt):
    return pl.pallas_call(
        _dinv_body,
        out_shape=jax.ShapeDtypeStruct((NPAD // F, F), jnp.float32),
    )(cnt)


def _mm_scale_body(h_ref, w_ref, dinv_ref, o_ref):
    hw = jnp.dot(h_ref[...], w_ref[...], preferred_element_type=jnp.float32)
    o_ref[...] = hw * dinv_ref[...]


def _mm_scale(h, w, dinv_col):
    return pl.pallas_call(
        _mm_scale_body,
        grid=(N // RBLK,),
        in_specs=[
            pl.BlockSpec((RBLK, F), lambda i: (i, 0)),
            pl.BlockSpec((F, F), lambda i: (0, 0)),
            pl.BlockSpec((RBLK, 1), lambda i: (i, 0)),
        ],
        out_specs=pl.BlockSpec((RBLK, F), lambda i: (i, 0)),
        out_shape=jax.ShapeDtypeStruct((N, F), jnp.float32),
    )(h, w, dinv_col)


def _combine_mm_body(pa_ref, pb_ref, hs_ref, dinv_ref, b_ref, w_ref, o_ref):
    a = dinv_ref[...] * (pa_ref[...] + pb_ref[...] - hs_ref[...]) + b_ref[...]
    t = jnp.maximum(a, 0.0)
    hw = jnp.dot(t, w_ref[...], preferred_element_type=jnp.float32)
    o_ref[...] = hw * dinv_ref[...]


def _combine_mm(pa, pb, hs, dinv_col, b_row, w):
    return pl.pallas_call(
        _combine_mm_body,
        grid=(N // RBLK,),
        in_specs=[
            pl.BlockSpec((RBLK, F), lambda i: (i, 0)),
            pl.BlockSpec((RBLK, F), lambda i: (i, 0)),
            pl.BlockSpec((RBLK, F), lambda i: (i, 0)),
            pl.BlockSpec((RBLK, 1), lambda i: (i, 0)),
            pl.BlockSpec((1, F), lambda i: (0, 0)),
            pl.BlockSpec((F, F), lambda i: (0, 0)),
        ],
        out_specs=pl.BlockSpec((RBLK, F), lambda i: (i, 0)),
        out_shape=jax.ShapeDtypeStruct((N, F), jnp.float32),
    )(pa, pb, hs, dinv_col, b_row, w)


def _final_body(pa_ref, pb_ref, hs_ref, dinv_ref, b_ref, o_ref):
    a = dinv_ref[...] * (pa_ref[...] + pb_ref[...] - hs_ref[...]) + b_ref[...]
    m = jnp.max(a, axis=1, keepdims=True)
    lse = jnp.log(jnp.sum(jnp.exp(a - m), axis=1, keepdims=True)) + m
    o_ref[...] = a - lse


def _final(pa, pb, hs, dinv_col, b_row):
    return pl.pallas_call(
        _final_body,
        grid=(N // RBLK,),
        in_specs=[
            pl.BlockSpec((RBLK, F), lambda i: (i, 0)),
            pl.BlockSpec((RBLK, F), lambda i: (i, 0)),
            pl.BlockSpec((RBLK, F), lambda i: (i, 0)),
            pl.BlockSpec((RBLK, 1), lambda i: (i, 0)),
            pl.BlockSpec((1, F), lambda i: (0, 0)),
        ],
        out_specs=pl.BlockSpec((RBLK, F), lambda i: (i, 0)),
        out_shape=jax.ShapeDtypeStruct((N, F), jnp.float32),
    )(pa, pb, hs, dinv_col, b_row)


_DBG_JNP_DEG = False
_DBG_JNP_AGG = False


def _jnp_deg(dst):
    c = jnp.zeros((NPAD,), jnp.float32).at[dst].add(1.0)
    return jnp.broadcast_to(c[None, :, None] * 0.5, (NC, NPAD, CW))


def _jnp_agg(hs, src, dst):
    a = hs.at[dst].add(hs[src], mode="promise_in_bounds")
    return jnp.stack([a, hs])


def kernel(x, adj, W1, b1, W2, b2, W3, b3):
    src = adj[0].astype(jnp.int32)
    dst = adj[1].astype(jnp.int32)

    if _DBG_JNP_DEG:
        cnt = _jnp_deg(dst)
    else:
        cnt = _deg_kernel(dst,
                          jnp.zeros((NPAD, CW), jnp.float32),
                          jnp.ones((CHUNK, CW), jnp.float32))
    cnt2d = cnt[:, :, 0].reshape(NC, NPAD // F, F)
    dinv_col = _dinv_kernel(cnt2d).reshape(NPAD, 1)[:N]

    agg = _jnp_agg if _DBG_JNP_AGG else (lambda h, s_, d_: _agg_kernel(h, s_, d_))

    hs = _mm_scale(x, W1, dinv_col)
    p = agg(hs, src, dst)
    hs = _combine_mm(p[0], p[1], hs, dinv_col, b1.reshape(1, F), W2)
    p = agg(hs, src, dst)
    hs = _combine_mm(p[0], p[1], hs, dinv_col, b2.reshape(1, F), W3)
    p = agg(hs, src, dst)
    return _final(p[0], p[1], hs, dinv_col, b3.reshape(1, F))


# speedup vs baseline: 20.1735x; 1.0441x over previous
import functools

import jax
import jax.numpy as jnp
from jax import lax
from jax.experimental import pallas as pl
from jax.experimental.pallas import tpu as pltpu
from jax.experimental.pallas import tpu_sc as plsc

N = 10000
E = 320000
F = 128
NPAD = 10240
CW = 128
CHUNK = 128
NCHUNKS = E // CHUNK
NC, NS = 2, 16
NW = NC * NS
IBLK = 400
NIBLK = N // IBLK
CNT_PER_TILE = NPAD // NS
RBLK = 1000

_mesh = plsc.VectorSubcoreMesh(core_axis_name="c", subcore_axis_name="s",
                               num_cores=NC, num_subcores=NS)


def _worker_id():
    return lax.axis_index("s") * NC + lax.axis_index("c")


NBUF = 3
NITER = 81
NCYC = NITER // NBUF


def _deg_body(dst_hbm, zeros_hbm, ones_hbm, out_hbm, dbufs, ones_v, acc,
              isems, ssems):
    c = lax.axis_index("c")
    s = lax.axis_index("s")
    wid = _worker_id()
    row0 = s * CNT_PER_TILE
    pltpu.sync_copy(zeros_hbm.at[pl.ds(row0, CNT_PER_TILE)],
                    acc.at[pl.ds(row0, CNT_PER_TILE)])
    pltpu.sync_copy(ones_hbm, ones_v)
    plsc.subcore_barrier()

    def idx_load(b, j):
        cid = j * NW + wid
        pltpu.async_copy(dst_hbm.at[pl.ds(cid * CHUNK, CHUNK)], dbufs[b],
                         isems[b])

    def idx_wait(b):
        pltpu.make_async_copy(dst_hbm.at[pl.ds(0, CHUNK)], dbufs[b],
                              isems[b]).wait()

    for b in range(NBUF):
        idx_load(b, b)

    def cycle(g, carry):
        for b in range(NBUF):
            cid = (g * NBUF + b) * NW + wid

            @pl.when(cid < NCHUNKS)
            def _(b=b):
                idx_wait(b)
                pltpu.async_copy(ones_v, acc.at[dbufs[b]], ssems[b], add=True)

        for b in range(NBUF):
            j = g * NBUF + b
            cid = j * NW + wid
            cidn = (j + NBUF) * NW + wid

            @pl.when(cid < NCHUNKS)
            def _(b=b):
                pltpu.make_async_copy(ones_v, acc.at[dbufs[b]],
                                      ssems[b]).wait()

            @pl.when(cidn < NCHUNKS)
            def _(b=b, j=j):
                idx_load(b, j + NBUF)

        return carry

    lax.fori_loop(0, NCYC, cycle, 0)
    plsc.subcore_barrier()
    pltpu.sync_copy(acc.at[pl.ds(row0, CNT_PER_TILE)],
                    out_hbm.at[c, pl.ds(row0, CNT_PER_TILE)])


_deg_kernel = functools.partial(
    pl.kernel,
    out_type=jax.ShapeDtypeStruct((NC, NPAD, CW), jnp.float32),
    mesh=_mesh,
    scratch_types=[
        [pltpu.VMEM((CHUNK,), jnp.int32) for _ in range(NBUF)],
        pltpu.VMEM((CHUNK, CW), jnp.float32),
        pltpu.VMEM_SHARED((NPAD, CW), jnp.float32),
        [pltpu.SemaphoreType.DMA for _ in range(NBUF)],
        [pltpu.SemaphoreType.DMA for _ in range(NBUF)],
    ],
)(_deg_body)


def _agg_body(hs_hbm, src_hbm, dst_hbm, out_hbm,
              sbufs, dbufs, rows, acc, isems, gsems, ssems):
    c = lax.axis_index("c")
    s = lax.axis_index("s")
    wid = _worker_id()

    def init_blk(j, carry):
        bid = j * NS + s

        @pl.when(bid < NIBLK)
        def _():
            pltpu.sync_copy(hs_hbm.at[pl.ds(bid * IBLK, IBLK)],
                            acc.at[pl.ds(bid * IBLK, IBLK)])

        return carry

    lax.fori_loop(0, (NIBLK + NS - 1) // NS, init_blk, 0)
    plsc.subcore_barrier()

    def idx_load(b, j):
        cid = j * NW + wid
        pltpu.async_copy(src_hbm.at[pl.ds(cid * CHUNK, CHUNK)], sbufs[b],
                         isems[b])
        pltpu.async_copy(dst_hbm.at[pl.ds(cid * CHUNK, CHUNK)], dbufs[b],
                         isems[b])

    def idx_wait(b):
        pltpu.make_async_copy(src_hbm.at[pl.ds(0, CHUNK)], sbufs[b],
                              isems[b]).wait()
        pltpu.make_async_copy(src_hbm.at[pl.ds(0, CHUNK)], dbufs[b],
                              isems[b]).wait()

    def gather(b):
        pltpu.async_copy(hs_hbm.at[sbufs[b]], rows[b], gsems[b])

    for b in range(NBUF):
        idx_load(b, b)
    for b in range(NBUF):
        idx_wait(b)
        gather(b)

    def cycle(g, carry):
        for b in range(NBUF):
            cid = (g * NBUF + b) * NW + wid

            @pl.when(cid < NCHUNKS)
            def _(b=b):
                pltpu.make_async_copy(hs_hbm.at[sbufs[b]], rows[b],
                                      gsems[b]).wait()
                pltpu.async_copy(rows[b], acc.at[dbufs[b]], ssems[b], add=True)

        for b in range(NBUF):
            j = g * NBUF + b
            cid = j * NW + wid
            cidn = (j + NBUF) * NW + wid

            @pl.when(cid < NCHUNKS)
            def _(b=b):
                pltpu.make_async_copy(rows[b], acc.at[dbufs[b]],
                                      ssems[b]).wait()

            @pl.when(cidn < NCHUNKS)
            def _(b=b, j=j):
                idx_load(b, j + NBUF)

        for b in range(NBUF):
            cidn = (g * NBUF + b + NBUF) * NW + wid

            @pl.when(cidn < NCHUNKS)
            def _(b=b):
                idx_wait(b)
                gather(b)

        return carry

    lax.fori_loop(0, NCYC, cycle, 0)
    plsc.subcore_barrier()

    def out_blk(j, carry):
        bid = j * NS + s

        @pl.when(bid < NIBLK)
        def _():
            pltpu.sync_copy(acc.at[pl.ds(bid * IBLK, IBLK)],
                            out_hbm.at[c, pl.ds(bid * IBLK, IBLK)])

        return carry

    lax.fori_loop(0, (NIBLK + NS - 1) // NS, out_blk, 0)


_agg_kernel = functools.partial(
    pl.kernel,
    out_type=jax.ShapeDtypeStruct((NC, N, F), jnp.float32),
    mesh=_mesh,
    scratch_types=[
        [pltpu.VMEM((CHUNK,), jnp.int32) for _ in range(NBUF)],
        [pltpu.VMEM((CHUNK,), jnp.int32) for _ in range(NBUF)],
        [pltpu.VMEM((CHUNK, F), jnp.float32) for _ in range(NBUF)],
        pltpu.VMEM_SHARED((N, F), jnp.float32),
        [pltpu.SemaphoreType.DMA for _ in range(NBUF)],
        [pltpu.SemaphoreType.DMA for _ in range(NBUF)],
        [pltpu.SemaphoreType.DMA for _ in range(NBUF)],
    ],
)(_agg_body)


def _dinv_body(cnt_ref, o_ref):
    deg = cnt_ref[0] + cnt_ref[1] + 1.0
    o_ref[...] = lax.rsqrt(deg)


def _dinv_kernel(cnt):
    return pl.pallas_call(
        _dinv_body,
        out_shape=jax.ShapeDtypeStruct((NPAD // F, F), jnp.float32),
    )(cnt)


def _mm_scale_body(h_ref, w_ref, dinv_ref, o_ref):
    hw = jnp.dot(h_ref[...], w_ref[...], preferred_element_type=jnp.float32)
    o_ref[...] = hw * dinv_ref[...]


def _mm_scale(h, w, dinv_col):
    return pl.pallas_call(
        _mm_scale_body,
        grid=(N // RBLK,),
        in_specs=[
            pl.BlockSpec((RBLK, F), lambda i: (i, 0)),
            pl.BlockSpec((F, F), lambda i: (0, 0)),
            pl.BlockSpec((RBLK, 1), lambda i: (i, 0)),
        ],
        out_specs=pl.BlockSpec((RBLK, F), lambda i: (i, 0)),
        out_shape=jax.ShapeDtypeStruct((N, F), jnp.float32),
    )(h, w, dinv_col)


def _combine_mm_body(pa_ref, pb_ref, hs_ref, dinv_ref, b_ref, w_ref, o_ref):
    a = dinv_ref[...] * (pa_ref[...] + pb_ref[...] - hs_ref[...]) + b_ref[...]
    t = jnp.maximum(a, 0.0)
    hw = jnp.dot(t, w_ref[...], preferred_element_type=jnp.float32)
    o_ref[...] = hw * dinv_ref[...]


def _combine_mm(pa, pb, hs, dinv_col, b_row, w):
    return pl.pallas_call(
        _combine_mm_body,
        grid=(N // RBLK,),
        in_specs=[
            pl.BlockSpec((RBLK, F), lambda i: (i, 0)),
            pl.BlockSpec((RBLK, F), lambda i: (i, 0)),
            pl.BlockSpec((RBLK, F), lambda i: (i, 0)),
            pl.BlockSpec((RBLK, 1), lambda i: (i, 0)),
            pl.BlockSpec((1, F), lambda i: (0, 0)),
            pl.BlockSpec((F, F), lambda i: (0, 0)),
        ],
        out_specs=pl.BlockSpec((RBLK, F), lambda i: (i, 0)),
        out_shape=jax.ShapeDtypeStruct((N, F), jnp.float32),
    )(pa, pb, hs, dinv_col, b_row, w)


def _final_body(pa_ref, pb_ref, hs_ref, dinv_ref, b_ref, o_ref):
    a = dinv_ref[...] * (pa_ref[...] + pb_ref[...] - hs_ref[...]) + b_ref[...]
    m = jnp.max(a, axis=1, keepdims=True)
    lse = jnp.log(jnp.sum(jnp.exp(a - m), axis=1, keepdims=True)) + m
    o_ref[...] = a - lse


def _final(pa, pb, hs, dinv_col, b_row):
    return pl.pallas_call(
        _final_body,
        grid=(N // RBLK,),
        in_specs=[
            pl.BlockSpec((RBLK, F), lambda i: (i, 0)),
            pl.BlockSpec((RBLK, F), lambda i: (i, 0)),
            pl.BlockSpec((RBLK, F), lambda i: (i, 0)),
            pl.BlockSpec((RBLK, 1), lambda i: (i, 0)),
            pl.BlockSpec((1, F), lambda i: (0, 0)),
        ],
        out_specs=pl.BlockSpec((RBLK, F), lambda i: (i, 0)),
        out_shape=jax.ShapeDtypeStruct((N, F), jnp.float32),
    )(pa, pb, hs, dinv_col, b_row)


_DBG_JNP_DEG = False
_DBG_JNP_AGG = False


def _jnp_deg(dst):
    c = jnp.zeros((NPAD,), jnp.float32).at[dst].add(1.0)
    return jnp.broadcast_to(c[None, :, None] * 0.5, (NC, NPAD, CW))


def _jnp_agg(hs, src, dst):
    a = hs.at[dst].add(hs[src], mode="promise_in_bounds")
    return jnp.stack([a, hs])


def kernel(x, adj, W1, b1, W2, b2, W3, b3):
    src = adj[0].astype(jnp.int32)
    dst = adj[1].astype(jnp.int32)

    if _DBG_JNP_DEG:
        cnt = _jnp_deg(dst)
    else:
        cnt = _deg_kernel(dst,
                          jnp.zeros((NPAD, CW), jnp.float32),
                          jnp.ones((CHUNK, CW), jnp.float32))
    cnt2d = cnt[:, :, 0].reshape(NC, NPAD // F, F)
    dinv_col = _dinv_kernel(cnt2d).reshape(NPAD, 1)[:N]

    agg = _jnp_agg if _DBG_JNP_AGG else (lambda h, s_, d_: _agg_kernel(h, s_, d_))

    hs = _mm_scale(x, W1, dinv_col)
    p = agg(hs, src, dst)
    hs = _combine_mm(p[0], p[1], hs, dinv_col, b1.reshape(1, F), W2)
    p = agg(hs, src, dst)
    hs = _combine_mm(p[0], p[1], hs, dinv_col, b2.reshape(1, F), W3)
    p = agg(hs, src, dst)
    return _final(p[0], p[1], hs, dinv_col, b3.reshape(1, F))


# speedup vs baseline: 20.8918x; 1.0356x over previous
import functools

import jax
import jax.numpy as jnp
from jax import lax
from jax.experimental import pallas as pl
from jax.experimental.pallas import tpu as pltpu
from jax.experimental.pallas import tpu_sc as plsc

N = 10000
E = 320000
F = 128
NPAD = 10240
CW = 128
CHUNK = 64
NCHUNKS = E // CHUNK
NC, NS = 2, 16
NW = NC * NS
IBLK = 400
NIBLK = N // IBLK
CNT_PER_TILE = NPAD // NS
RBLK = 1000

_mesh = plsc.VectorSubcoreMesh(core_axis_name="c", subcore_axis_name="s",
                               num_cores=NC, num_subcores=NS)


def _worker_id():
    return lax.axis_index("s") * NC + lax.axis_index("c")


NBUF = 6
NITER = 162
NCYC = NITER // NBUF


def _deg_body(dst_hbm, zeros_hbm, ones_hbm, out_hbm, dbufs, ones_v, acc,
              isems, ssems):
    c = lax.axis_index("c")
    s = lax.axis_index("s")
    wid = _worker_id()
    row0 = s * CNT_PER_TILE
    pltpu.sync_copy(zeros_hbm.at[pl.ds(row0, CNT_PER_TILE)],
                    acc.at[pl.ds(row0, CNT_PER_TILE)])
    pltpu.sync_copy(ones_hbm, ones_v)
    plsc.subcore_barrier()

    def idx_load(b, j):
        cid = j * NW + wid
        pltpu.async_copy(dst_hbm.at[pl.ds(cid * CHUNK, CHUNK)], dbufs[b],
                         isems[b])

    def idx_wait(b):
        pltpu.make_async_copy(dst_hbm.at[pl.ds(0, CHUNK)], dbufs[b],
                              isems[b]).wait()

    for b in range(NBUF):
        idx_load(b, b)

    def cycle(g, carry):
        for b in range(NBUF):
            cid = (g * NBUF + b) * NW + wid

            @pl.when(cid < NCHUNKS)
            def _(b=b):
                idx_wait(b)
                pltpu.async_copy(ones_v, acc.at[dbufs[b]], ssems[b], add=True)

        for b in range(NBUF):
            j = g * NBUF + b
            cid = j * NW + wid
            cidn = (j + NBUF) * NW + wid

            @pl.when(cid < NCHUNKS)
            def _(b=b):
                pltpu.make_async_copy(ones_v, acc.at[dbufs[b]],
                                      ssems[b]).wait()

            @pl.when(cidn < NCHUNKS)
            def _(b=b, j=j):
                idx_load(b, j + NBUF)

        return carry

    lax.fori_loop(0, NCYC, cycle, 0)
    plsc.subcore_barrier()
    pltpu.sync_copy(acc.at[pl.ds(row0, CNT_PER_TILE)],
                    out_hbm.at[c, pl.ds(row0, CNT_PER_TILE)])


_deg_kernel = functools.partial(
    pl.kernel,
    out_type=jax.ShapeDtypeStruct((NC, NPAD, CW), jnp.float32),
    mesh=_mesh,
    scratch_types=[
        [pltpu.VMEM((CHUNK,), jnp.int32) for _ in range(NBUF)],
        pltpu.VMEM((CHUNK, CW), jnp.float32),
        pltpu.VMEM_SHARED((NPAD, CW), jnp.float32),
        [pltpu.SemaphoreType.DMA for _ in range(NBUF)],
        [pltpu.SemaphoreType.DMA for _ in range(NBUF)],
    ],
)(_deg_body)


def _agg_body(hs_hbm, src_hbm, dst_hbm, out_hbm,
              sbufs, dbufs, rows, acc, isems, gsems, ssems):
    c = lax.axis_index("c")
    s = lax.axis_index("s")
    wid = _worker_id()

    def init_blk(j, carry):
        bid = j * NS + s

        @pl.when(bid < NIBLK)
        def _():
            pltpu.sync_copy(hs_hbm.at[pl.ds(bid * IBLK, IBLK)],
                            acc.at[pl.ds(bid * IBLK, IBLK)])

        return carry

    lax.fori_loop(0, (NIBLK + NS - 1) // NS, init_blk, 0)
    plsc.subcore_barrier()

    def idx_load(b, j):
        cid = j * NW + wid
        pltpu.async_copy(src_hbm.at[pl.ds(cid * CHUNK, CHUNK)], sbufs[b],
                         isems[b])
        pltpu.async_copy(dst_hbm.at[pl.ds(cid * CHUNK, CHUNK)], dbufs[b],
                         isems[b])

    def idx_wait(b):
        pltpu.make_async_copy(src_hbm.at[pl.ds(0, CHUNK)], sbufs[b],
                              isems[b]).wait()
        pltpu.make_async_copy(src_hbm.at[pl.ds(0, CHUNK)], dbufs[b],
                              isems[b]).wait()

    def gather(b):
        pltpu.async_copy(hs_hbm.at[sbufs[b]], rows[b], gsems[b])

    for b in range(NBUF):
        idx_load(b, b)
    for b in range(NBUF):
        idx_wait(b)
        gather(b)

    def cycle(g, carry):
        for b in range(NBUF):
            cid = (g * NBUF + b) * NW + wid

            @pl.when(cid < NCHUNKS)
            def _(b=b):
                pltpu.make_async_copy(hs_hbm.at[sbufs[b]], rows[b],
                                      gsems[b]).wait()
                pltpu.async_copy(rows[b], acc.at[dbufs[b]], ssems[b], add=True)

        for b in range(NBUF):
            j = g * NBUF + b
            cid = j * NW + wid
            cidn = (j + NBUF) * NW + wid

            @pl.when(cid < NCHUNKS)
            def _(b=b):
                pltpu.make_async_copy(rows[b], acc.at[dbufs[b]],
                                      ssems[b]).wait()

            @pl.when(cidn < NCHUNKS)
            def _(b=b, j=j):
                idx_load(b, j + NBUF)

        for b in range(NBUF):
            cidn = (g * NBUF + b + NBUF) * NW + wid

            @pl.when(cidn < NCHUNKS)
            def _(b=b):
                idx_wait(b)
                gather(b)

        return carry

    lax.fori_loop(0, NCYC, cycle, 0)
    plsc.subcore_barrier()

    def out_blk(j, carry):
        bid = j * NS + s

        @pl.when(bid < NIBLK)
        def _():
            pltpu.sync_copy(acc.at[pl.ds(bid * IBLK, IBLK)],
                            out_hbm.at[c, pl.ds(bid * IBLK, IBLK)])

        return carry

    lax.fori_loop(0, (NIBLK + NS - 1) // NS, out_blk, 0)


_agg_kernel = functools.partial(
    pl.kernel,
    out_type=jax.ShapeDtypeStruct((NC, N, F), jnp.float32),
    mesh=_mesh,
    scratch_types=[
        [pltpu.VMEM((CHUNK,), jnp.int32) for _ in range(NBUF)],
        [pltpu.VMEM((CHUNK,), jnp.int32) for _ in range(NBUF)],
        [pltpu.VMEM((CHUNK, F), jnp.float32) for _ in range(NBUF)],
        pltpu.VMEM_SHARED((N, F), jnp.float32),
        [pltpu.SemaphoreType.DMA for _ in range(NBUF)],
        [pltpu.SemaphoreType.DMA for _ in range(NBUF)],
        [pltpu.SemaphoreType.DMA for _ in range(NBUF)],
    ],
)(_agg_body)


def _dinv_body(cnt_ref, o_ref):
    deg = cnt_ref[0] + cnt_ref[1] + 1.0
    o_ref[...] = lax.rsqrt(deg)


def _dinv_kernel(cnt):
    return pl.pallas_call(
        _dinv_body,
        out_shape=jax.ShapeDtypeStruct((NPAD // F, F), jnp.float32),
    )(cnt)


def _mm_scale_body(h_ref, w_ref, dinv_ref, o_ref):
    hw = jnp.dot(h_ref[...], w_ref[...], preferred_element_type=jnp.float32)
    o_ref[...] = hw * dinv_ref[...]


def _mm_scale(h, w, dinv_col):
    return pl.pallas_call(
        _mm_scale_body,
        grid=(N // RBLK,),
        in_specs=[
            pl.BlockSpec((RBLK, F), lambda i: (i, 0)),
            pl.BlockSpec((F, F), lambda i: (0, 0)),
            pl.BlockSpec((RBLK, 1), lambda i: (i, 0)),
        ],
        out_specs=pl.BlockSpec((RBLK, F), lambda i: (i, 0)),
        out_shape=jax.ShapeDtypeStruct((N, F), jnp.float32),
    )(h, w, dinv_col)


def _combine_mm_body(pa_ref, pb_ref, hs_ref, dinv_ref, b_ref, w_ref, o_ref):
    a = dinv_ref[...] * (pa_ref[...] + pb_ref[...] - hs_ref[...]) + b_ref[...]
    t = jnp.maximum(a, 0.0)
    hw = jnp.dot(t, w_ref[...], preferred_element_type=jnp.float32)
    o_ref[...] = hw * dinv_ref[...]


def _combine_mm(pa, pb, hs, dinv_col, b_row, w):
    return pl.pallas_call(
        _combine_mm_body,
        grid=(N // RBLK,),
        in_specs=[
            pl.BlockSpec((RBLK, F), lambda i: (i, 0)),
            pl.BlockSpec((RBLK, F), lambda i: (i, 0)),
            pl.BlockSpec((RBLK, F), lambda i: (i, 0)),
            pl.BlockSpec((RBLK, 1), lambda i: (i, 0)),
            pl.BlockSpec((1, F), lambda i: (0, 0)),
            pl.BlockSpec((F, F), lambda i: (0, 0)),
        ],
        out_specs=pl.BlockSpec((RBLK, F), lambda i: (i, 0)),
        out_shape=jax.ShapeDtypeStruct((N, F), jnp.float32),
    )(pa, pb, hs, dinv_col, b_row, w)


def _final_body(pa_ref, pb_ref, hs_ref, dinv_ref, b_ref, o_ref):
    a = dinv_ref[...] * (pa_ref[...] + pb_ref[...] - hs_ref[...]) + b_ref[...]
    m = jnp.max(a, axis=1, keepdims=True)
    lse = jnp.log(jnp.sum(jnp.exp(a - m), axis=1, keepdims=True)) + m
    o_ref[...] = a - lse


def _final(pa, pb, hs, dinv_col, b_row):
    return pl.pallas_call(
        _final_body,
        grid=(N // RBLK,),
        in_specs=[
            pl.BlockSpec((RBLK, F), lambda i: (i, 0)),
            pl.BlockSpec((RBLK, F), lambda i: (i, 0)),
            pl.BlockSpec((RBLK, F), lambda i: (i, 0)),
            pl.BlockSpec((RBLK, 1), lambda i: (i, 0)),
            pl.BlockSpec((1, F), lambda i: (0, 0)),
        ],
        out_specs=pl.BlockSpec((RBLK, F), lambda i: (i, 0)),
        out_shape=jax.ShapeDtypeStruct((N, F), jnp.float32),
    )(pa, pb, hs, dinv_col, b_row)


_DBG_JNP_DEG = False
_DBG_JNP_AGG = False


def _jnp_deg(dst):
    c = jnp.zeros((NPAD,), jnp.float32).at[dst].add(1.0)
    return jnp.broadcast_to(c[None, :, None] * 0.5, (NC, NPAD, CW))


def _jnp_agg(hs, src, dst):
    a = hs.at[dst].add(hs[src], mode="promise_in_bounds")
    return jnp.stack([a, hs])


def kernel(x, adj, W1, b1, W2, b2, W3, b3):
    src = adj[0].astype(jnp.int32)
    dst = adj[1].astype(jnp.int32)

    if _DBG_JNP_DEG:
        cnt = _jnp_deg(dst)
    else:
        cnt = _deg_kernel(dst,
                          jnp.zeros((NPAD, CW), jnp.float32),
                          jnp.ones((CHUNK, CW), jnp.float32))
    cnt2d = cnt[:, :, 0].reshape(NC, NPAD // F, F)
    dinv_col = _dinv_kernel(cnt2d).reshape(NPAD, 1)[:N]

    agg = _jnp_agg if _DBG_JNP_AGG else (lambda h, s_, d_: _agg_kernel(h, s_, d_))

    hs = _mm_scale(x, W1, dinv_col)
    p = agg(hs, src, dst)
    hs = _combine_mm(p[0], p[1], hs, dinv_col, b1.reshape(1, F), W2)
    p = agg(hs, src, dst)
    hs = _combine_mm(p[0], p[1], hs, dinv_col, b2.reshape(1, F), W3)
    p = agg(hs, src, dst)
    return _final(p[0], p[1], hs, dinv_col, b3.reshape(1, F))


# speedup vs baseline: 21.0983x; 1.0099x over previous
import functools

import jax
import jax.numpy as jnp
from jax import lax
from jax.experimental import pallas as pl
from jax.experimental.pallas import tpu as pltpu
from jax.experimental.pallas import tpu_sc as plsc

N = 10000
E = 320000
F = 128
NPAD = 10240
CW = 128
CHUNK = 64
NCHUNKS = E // CHUNK
NC, NS = 2, 16
NW = NC * NS
IBLK = 400
NIBLK = N // IBLK
CNT_PER_TILE = NPAD // NS
RBLK = 1000

_mesh = plsc.VectorSubcoreMesh(core_axis_name="c", subcore_axis_name="s",
                               num_cores=NC, num_subcores=NS)


def _worker_id():
    return lax.axis_index("s") * NC + lax.axis_index("c")


NBUF = 6
NITER = 162
NCYC = NITER // NBUF


def _deg_body(dst_hbm, zeros_hbm, ones_hbm, out_hbm, dbufs, ones_v, acc,
              isems, ssems):
    c = lax.axis_index("c")
    s = lax.axis_index("s")
    wid = _worker_id()
    row0 = s * CNT_PER_TILE
    pltpu.sync_copy(zeros_hbm.at[pl.ds(row0, CNT_PER_TILE)],
                    acc.at[pl.ds(row0, CNT_PER_TILE)])
    pltpu.sync_copy(ones_hbm, ones_v)
    plsc.subcore_barrier()

    def idx_load(b, j):
        cid = j * NW + wid
        pltpu.async_copy(dst_hbm.at[pl.ds(cid * CHUNK, CHUNK)], dbufs[b],
                         isems[b])

    def idx_wait(b):
        pltpu.make_async_copy(dst_hbm.at[pl.ds(0, CHUNK)], dbufs[b],
                              isems[b]).wait()

    for b in range(NBUF):
        idx_load(b, b)

    def cycle(g, carry):
        for b in range(NBUF):
            cid = (g * NBUF + b) * NW + wid

            @pl.when(cid < NCHUNKS)
            def _(b=b):
                idx_wait(b)
                pltpu.async_copy(ones_v, acc.at[dbufs[b]], ssems[b], add=True)

        for b in range(NBUF):
            j = g * NBUF + b
            cid = j * NW + wid
            cidn = (j + NBUF) * NW + wid

            @pl.when(cid < NCHUNKS)
            def _(b=b):
                pltpu.make_async_copy(ones_v, acc.at[dbufs[b]],
                                      ssems[b]).wait()

            @pl.when(cidn < NCHUNKS)
            def _(b=b, j=j):
                idx_load(b, j + NBUF)

        return carry

    lax.fori_loop(0, NCYC, cycle, 0)
    plsc.subcore_barrier()
    pltpu.sync_copy(acc.at[pl.ds(row0, CNT_PER_TILE)],
                    out_hbm.at[c, pl.ds(row0, CNT_PER_TILE)])


_deg_kernel = functools.partial(
    pl.kernel,
    out_type=jax.ShapeDtypeStruct((NC, NPAD, CW), jnp.float32),
    mesh=_mesh,
    scratch_types=[
        [pltpu.VMEM((CHUNK,), jnp.int32) for _ in range(NBUF)],
        pltpu.VMEM((CHUNK, CW), jnp.float32),
        pltpu.VMEM_SHARED((NPAD, CW), jnp.float32),
        [pltpu.SemaphoreType.DMA for _ in range(NBUF)],
        [pltpu.SemaphoreType.DMA for _ in range(NBUF)],
    ],
)(_deg_body)


def _agg_body(hs_hbm, src_hbm, dst_hbm, out_hbm,
              sbufs, dbufs, rows, acc, isems, gsems, ssems):
    c = lax.axis_index("c")
    s = lax.axis_index("s")
    wid = _worker_id()

    def init_blk(j, carry):
        bid = j * NS + s

        @pl.when(bid < NIBLK)
        def _():
            pltpu.sync_copy(hs_hbm.at[pl.ds(bid * IBLK, IBLK)],
                            acc.at[pl.ds(bid * IBLK, IBLK)])

        return carry

    lax.fori_loop(0, (NIBLK + NS - 1) // NS, init_blk, 0)
    plsc.subcore_barrier()

    def idx_load(b, j):
        cid = j * NW + wid
        pltpu.async_copy(src_hbm.at[pl.ds(cid * CHUNK, CHUNK)], sbufs[b],
                         isems[b])
        pltpu.async_copy(dst_hbm.at[pl.ds(cid * CHUNK, CHUNK)], dbufs[b],
                         isems[b])

    def idx_wait(b):
        pltpu.make_async_copy(src_hbm.at[pl.ds(0, CHUNK)], sbufs[b],
                              isems[b]).wait()
        pltpu.make_async_copy(src_hbm.at[pl.ds(0, CHUNK)], dbufs[b],
                              isems[b]).wait()

    def gather(b):
        pltpu.async_copy(hs_hbm.at[sbufs[b]], rows[b], gsems[b])

    for b in range(NBUF):
        idx_load(b, b)
    for b in range(NBUF):
        idx_wait(b)
        gather(b)

    def cycle(g, carry):
        for b in range(NBUF):
            cid = (g * NBUF + b) * NW + wid

            @pl.when(cid < NCHUNKS)
            def _(b=b):
                pltpu.make_async_copy(hs_hbm.at[sbufs[b]], rows[b],
                                      gsems[b]).wait()
                pltpu.async_copy(rows[b], acc.at[dbufs[b]], ssems[b], add=True)

        for b in range(NBUF):
            j = g * NBUF + b
            cid = j * NW + wid
            cidn = (j + NBUF) * NW + wid

            @pl.when(cid < NCHUNKS)
            def _(b=b):
                pltpu.make_async_copy(rows[b], acc.at[dbufs[b]],
                                      ssems[b]).wait()

            @pl.when(cidn < NCHUNKS)
            def _(b=b, j=j):
                idx_load(b, j + NBUF)

        for b in range(NBUF):
            cidn = (g * NBUF + b + NBUF) * NW + wid

            @pl.when(cidn < NCHUNKS)
            def _(b=b):
                idx_wait(b)
                gather(b)

        return carry

    lax.fori_loop(0, NCYC, cycle, 0)
    plsc.subcore_barrier()

    def out_blk(j, carry):
        bid = j * NS + s

        @pl.when(bid < NIBLK)
        def _():
            pltpu.sync_copy(acc.at[pl.ds(bid * IBLK, IBLK)],
                            out_hbm.at[c, pl.ds(bid * IBLK, IBLK)])

        return carry

    lax.fori_loop(0, (NIBLK + NS - 1) // NS, out_blk, 0)


_agg_kernel = functools.partial(
    pl.kernel,
    out_type=jax.ShapeDtypeStruct((NC, N, F), jnp.float32),
    mesh=_mesh,
    scratch_types=[
        [pltpu.VMEM((CHUNK,), jnp.int32) for _ in range(NBUF)],
        [pltpu.VMEM((CHUNK,), jnp.int32) for _ in range(NBUF)],
        [pltpu.VMEM((CHUNK, F), jnp.float32) for _ in range(NBUF)],
        pltpu.VMEM_SHARED((N, F), jnp.float32),
        [pltpu.SemaphoreType.DMA for _ in range(NBUF)],
        [pltpu.SemaphoreType.DMA for _ in range(NBUF)],
        [pltpu.SemaphoreType.DMA for _ in range(NBUF)],
    ],
)(_agg_body)


def _mm_scale_body(h_ref, w_ref, cnt_ref, o_ref):
    dinv = lax.rsqrt(cnt_ref[...] + 1.0)
    hw = jnp.dot(h_ref[...], w_ref[...], preferred_element_type=jnp.float32)
    o_ref[...] = hw * dinv


def _mm_scale(h, w, dinv_col):
    return pl.pallas_call(
        _mm_scale_body,
        grid=(N // RBLK,),
        in_specs=[
            pl.BlockSpec((RBLK, F), lambda i: (i, 0)),
            pl.BlockSpec((F, F), lambda i: (0, 0)),
            pl.BlockSpec((RBLK, 1), lambda i: (i, 0)),
        ],
        out_specs=pl.BlockSpec((RBLK, F), lambda i: (i, 0)),
        out_shape=jax.ShapeDtypeStruct((N, F), jnp.float32),
    )(h, w, dinv_col)


def _combine_mm_body(pa_ref, pb_ref, hs_ref, cnt_ref, b_ref, w_ref, o_ref):
    dinv = lax.rsqrt(cnt_ref[...] + 1.0)
    a = dinv * (pa_ref[...] + pb_ref[...] - hs_ref[...]) + b_ref[...]
    t = jnp.maximum(a, 0.0)
    hw = jnp.dot(t, w_ref[...], preferred_element_type=jnp.float32)
    o_ref[...] = hw * dinv


def _combine_mm(pa, pb, hs, dinv_col, b_row, w):
    return pl.pallas_call(
        _combine_mm_body,
        grid=(N // RBLK,),
        in_specs=[
            pl.BlockSpec((RBLK, F), lambda i: (i, 0)),
            pl.BlockSpec((RBLK, F), lambda i: (i, 0)),
            pl.BlockSpec((RBLK, F), lambda i: (i, 0)),
            pl.BlockSpec((RBLK, 1), lambda i: (i, 0)),
            pl.BlockSpec((1, F), lambda i: (0, 0)),
            pl.BlockSpec((F, F), lambda i: (0, 0)),
        ],
        out_specs=pl.BlockSpec((RBLK, F), lambda i: (i, 0)),
        out_shape=jax.ShapeDtypeStruct((N, F), jnp.float32),
    )(pa, pb, hs, dinv_col, b_row, w)


def _final_body(pa_ref, pb_ref, hs_ref, cnt_ref, b_ref, o_ref):
    dinv = lax.rsqrt(cnt_ref[...] + 1.0)
    a = dinv * (pa_ref[...] + pb_ref[...] - hs_ref[...]) + b_ref[...]
    m = jnp.max(a, axis=1, keepdims=True)
    lse = jnp.log(jnp.sum(jnp.exp(a - m), axis=1, keepdims=True)) + m
    o_ref[...] = a - lse


def _final(pa, pb, hs, dinv_col, b_row):
    return pl.pallas_call(
        _final_body,
        grid=(N // RBLK,),
        in_specs=[
            pl.BlockSpec((RBLK, F), lambda i: (i, 0)),
            pl.BlockSpec((RBLK, F), lambda i: (i, 0)),
            pl.BlockSpec((RBLK, F), lambda i: (i, 0)),
            pl.BlockSpec((RBLK, 1), lambda i: (i, 0)),
            pl.BlockSpec((1, F), lambda i: (0, 0)),
        ],
        out_specs=pl.BlockSpec((RBLK, F), lambda i: (i, 0)),
        out_shape=jax.ShapeDtypeStruct((N, F), jnp.float32),
    )(pa, pb, hs, dinv_col, b_row)


_DBG_JNP_DEG = False
_DBG_JNP_AGG = False


def _jnp_deg(dst):
    c = jnp.zeros((NPAD,), jnp.float32).at[dst].add(1.0)
    return jnp.broadcast_to(c[None, :, None] * 0.5, (NC, NPAD, CW))


def _jnp_agg(hs, src, dst):
    a = hs.at[dst].add(hs[src], mode="promise_in_bounds")
    return jnp.stack([a, hs])


def kernel(x, adj, W1, b1, W2, b2, W3, b3):
    src = adj[0].astype(jnp.int32)
    dst = adj[1].astype(jnp.int32)

    if _DBG_JNP_DEG:
        cnt = _jnp_deg(dst)
    else:
        cnt = _deg_kernel(dst,
                          jnp.zeros((NPAD, CW), jnp.float32),
                          jnp.ones((CHUNK, CW), jnp.float32))
    dinv_col = (cnt[0, :N, 0] + cnt[1, :N, 0]).reshape(N, 1)

    agg = _jnp_agg if _DBG_JNP_AGG else (lambda h, s_, d_: _agg_kernel(h, s_, d_))

    hs = _mm_scale(x, W1, dinv_col)
    p = agg(hs, src, dst)
    hs = _combine_mm(p[0], p[1], hs, dinv_col, b1.reshape(1, F), W2)
    p = agg(hs, src, dst)
    hs = _combine_mm(p[0], p[1], hs, dinv_col, b2.reshape(1, F), W3)
    p = agg(hs, src, dst)
    return _final(p[0], p[1], hs, dinv_col, b3.reshape(1, F))


# speedup vs baseline: 21.1087x; 1.0005x over previous
import functools

import jax
import jax.numpy as jnp
from jax import lax
from jax.experimental import pallas as pl
from jax.experimental.pallas import tpu as pltpu
from jax.experimental.pallas import tpu_sc as plsc

N = 10000
E = 320000
F = 128
NPAD = 10240
CW = 128
CHUNK = 64
NCHUNKS = E // CHUNK
NC, NS = 2, 16
NW = NC * NS
IBLK = 400
NIBLK = N // IBLK
CNT_PER_TILE = NPAD // NS
RBLK = 1000

_mesh = plsc.VectorSubcoreMesh(core_axis_name="c", subcore_axis_name="s",
                               num_cores=NC, num_subcores=NS)


def _worker_id():
    return lax.axis_index("s") * NC + lax.axis_index("c")


NBUF = 6
NITER = 162
NCYC = NITER // NBUF


def _deg_body(dst_hbm, zeros_hbm, ones_hbm, out_hbm, dbufs, ones_v, acc,
              isems, ssems):
    c = lax.axis_index("c")
    s = lax.axis_index("s")
    wid = _worker_id()
    row0 = s * CNT_PER_TILE
    pltpu.sync_copy(zeros_hbm.at[pl.ds(row0, CNT_PER_TILE)],
                    acc.at[pl.ds(row0, CNT_PER_TILE)])
    pltpu.sync_copy(ones_hbm, ones_v)
    plsc.subcore_barrier()

    def idx_load(b, j):
        cid = j * NW + wid
        pltpu.async_copy(dst_hbm.at[pl.ds(cid * CHUNK, CHUNK)], dbufs[b],
                         isems[b])

    def idx_wait(b):
        pltpu.make_async_copy(dst_hbm.at[pl.ds(0, CHUNK)], dbufs[b],
                              isems[b]).wait()

    for b in range(NBUF):
        idx_load(b, b)

    def cycle(g, carry):
        for b in range(NBUF):
            cid = (g * NBUF + b) * NW + wid

            @pl.when(cid < NCHUNKS)
            def _(b=b):
                idx_wait(b)
                pltpu.async_copy(ones_v, acc.at[dbufs[b]], ssems[b], add=True)

        for b in range(NBUF):
            j = g * NBUF + b
            cid = j * NW + wid
            cidn = (j + NBUF) * NW + wid

            @pl.when(cid < NCHUNKS)
            def _(b=b):
                pltpu.make_async_copy(ones_v, acc.at[dbufs[b]],
                                      ssems[b]).wait()

            @pl.when(cidn < NCHUNKS)
            def _(b=b, j=j):
                idx_load(b, j + NBUF)

        return carry

    lax.fori_loop(0, NCYC, cycle, 0)
    plsc.subcore_barrier()
    pltpu.sync_copy(acc.at[pl.ds(row0, CNT_PER_TILE)],
                    out_hbm.at[c, pl.ds(row0, CNT_PER_TILE)])


_deg_kernel = functools.partial(
    pl.kernel,
    out_type=jax.ShapeDtypeStruct((NC, NPAD, CW), jnp.float32),
    mesh=_mesh,
    scratch_types=[
        [pltpu.VMEM((CHUNK,), jnp.int32) for _ in range(NBUF)],
        pltpu.VMEM((CHUNK, CW), jnp.float32),
        pltpu.VMEM_SHARED((NPAD, CW), jnp.float32),
        [pltpu.SemaphoreType.DMA for _ in range(NBUF)],
        [pltpu.SemaphoreType.DMA for _ in range(NBUF)],
    ],
)(_deg_body)


def _agg_body(hs_hbm, src_hbm, dst_hbm, out_hbm,
              sbufs, dbufs, rows, acc, isems, gsems, ssems):
    c = lax.axis_index("c")
    s = lax.axis_index("s")
    wid = _worker_id()

    def init_blk(j, carry):
        bid = j * NS + s

        @pl.when(bid < NIBLK)
        def _():
            pltpu.sync_copy(hs_hbm.at[pl.ds(bid * IBLK, IBLK)],
                            acc.at[pl.ds(bid * IBLK, IBLK)])

        return carry

    lax.fori_loop(0, (NIBLK + NS - 1) // NS, init_blk, 0)
    plsc.subcore_barrier()

    def idx_load(b, j):
        cid = j * NW + wid
        pltpu.async_copy(src_hbm.at[pl.ds(cid * CHUNK, CHUNK)], sbufs[b],
                         isems[b])
        pltpu.async_copy(dst_hbm.at[pl.ds(cid * CHUNK, CHUNK)], dbufs[b],
                         isems[b])

    def idx_wait(b):
        pltpu.make_async_copy(src_hbm.at[pl.ds(0, CHUNK)], sbufs[b],
                              isems[b]).wait()
        pltpu.make_async_copy(src_hbm.at[pl.ds(0, CHUNK)], dbufs[b],
                              isems[b]).wait()

    def gather(b):
        pltpu.async_copy(hs_hbm.at[sbufs[b]], rows[b], gsems[b])

    for b in range(NBUF):
        idx_load(b, b)
    for b in range(NBUF):
        idx_wait(b)
        gather(b)

    def cycle(g, carry):
        for b in range(NBUF):
            cid = (g * NBUF + b) * NW + wid

            @pl.when(cid < NCHUNKS)
            def _(b=b):
                pltpu.make_async_copy(hs_hbm.at[sbufs[b]], rows[b],
                                      gsems[b]).wait()
                pltpu.async_copy(rows[b], acc.at[dbufs[b]], ssems[b], add=True)

        for b in range(NBUF):
            j = g * NBUF + b
            cid = j * NW + wid
            cidn = (j + NBUF) * NW + wid

            @pl.when(cid < NCHUNKS)
            def _(b=b):
                pltpu.make_async_copy(rows[b], acc.at[dbufs[b]],
                                      ssems[b]).wait()

            @pl.when(cidn < NCHUNKS)
            def _(b=b, j=j):
                idx_load(b, j + NBUF)

        for b in range(NBUF):
            cidn = (g * NBUF + b + NBUF) * NW + wid

            @pl.when(cidn < NCHUNKS)
            def _(b=b):
                idx_wait(b)
                gather(b)

        return carry

    lax.fori_loop(0, NCYC, cycle, 0)
    plsc.subcore_barrier()

    def out_blk(j, carry):
        bid = j * NS + s

        @pl.when(bid < NIBLK)
        def _():
            pltpu.sync_copy(acc.at[pl.ds(bid * IBLK, IBLK)],
                            out_hbm.at[c, pl.ds(bid * IBLK, IBLK)])

        return carry

    lax.fori_loop(0, (NIBLK + NS - 1) // NS, out_blk, 0)


_agg_kernel = functools.partial(
    pl.kernel,
    out_type=jax.ShapeDtypeStruct((NC, N, F), jnp.float32),
    mesh=_mesh,
    scratch_types=[
        [pltpu.VMEM((CHUNK,), jnp.int32) for _ in range(NBUF)],
        [pltpu.VMEM((CHUNK,), jnp.int32) for _ in range(NBUF)],
        [pltpu.VMEM((CHUNK, F), jnp.float32) for _ in range(NBUF)],
        pltpu.VMEM_SHARED((N, F), jnp.float32),
        [pltpu.SemaphoreType.DMA for _ in range(NBUF)],
        [pltpu.SemaphoreType.DMA for _ in range(NBUF)],
        [pltpu.SemaphoreType.DMA for _ in range(NBUF)],
    ],
)(_agg_body)


def _mm_scale_body(h_ref, w_ref, cnt_ref, o_ref):
    dinv = lax.rsqrt(cnt_ref[...] + 1.0)
    hw = jnp.dot(h_ref[...], w_ref[...], preferred_element_type=jnp.float32)
    o_ref[...] = hw * dinv


def _mm_scale(h, w, dinv_col):
    return pl.pallas_call(
        _mm_scale_body,
        grid=(N // RBLK,),
        in_specs=[
            pl.BlockSpec((RBLK, F), lambda i: (i, 0)),
            pl.BlockSpec((F, F), lambda i: (0, 0)),
            pl.BlockSpec((RBLK, 1), lambda i: (i, 0)),
        ],
        out_specs=pl.BlockSpec((RBLK, F), lambda i: (i, 0)),
        out_shape=jax.ShapeDtypeStruct((N, F), jnp.float32),
    )(h, w, dinv_col)


def _combine_mm_body(pa_ref, pb_ref, hs_ref, cnt_ref, b_ref, w_ref, o_ref):
    dinv = lax.rsqrt(cnt_ref[...] + 1.0)
    a = dinv * (pa_ref[...] + pb_ref[...] - hs_ref[...]) + b_ref[...]
    t = jnp.maximum(a, 0.0)
    hw = jnp.dot(t, w_ref[...], preferred_element_type=jnp.float32)
    o_ref[...] = hw * dinv


def _combine_mm(pa, pb, hs, dinv_col, b_row, w):
    return pl.pallas_call(
        _combine_mm_body,
        grid=(N // RBLK,),
        in_specs=[
            pl.BlockSpec((RBLK, F), lambda i: (i, 0)),
            pl.BlockSpec((RBLK, F), lambda i: (i, 0)),
            pl.BlockSpec((RBLK, F), lambda i: (i, 0)),
            pl.BlockSpec((RBLK, 1), lambda i: (i, 0)),
            pl.BlockSpec((1, F), lambda i: (0, 0)),
            pl.BlockSpec((F, F), lambda i: (0, 0)),
        ],
        out_specs=pl.BlockSpec((RBLK, F), lambda i: (i, 0)),
        out_shape=jax.ShapeDtypeStruct((N, F), jnp.float32),
    )(pa, pb, hs, dinv_col, b_row, w)


def _final_body(pa_ref, pb_ref, hs_ref, cnt_ref, b_ref, o_ref):
    dinv = lax.rsqrt(cnt_ref[...] + 1.0)
    a = dinv * (pa_ref[...] + pb_ref[...] - hs_ref[...]) + b_ref[...]
    m = jnp.max(a, axis=1, keepdims=True)
    lse = jnp.log(jnp.sum(jnp.exp(a - m), axis=1, keepdims=True)) + m
    o_ref[...] = a - lse


def _final(pa, pb, hs, dinv_col, b_row):
    return pl.pallas_call(
        _final_body,
        grid=(N // RBLK,),
        in_specs=[
            pl.BlockSpec((RBLK, F), lambda i: (i, 0)),
            pl.BlockSpec((RBLK, F), lambda i: (i, 0)),
            pl.BlockSpec((RBLK, F), lambda i: (i, 0)),
            pl.BlockSpec((RBLK, 1), lambda i: (i, 0)),
            pl.BlockSpec((1, F), lambda i: (0, 0)),
        ],
        out_specs=pl.BlockSpec((RBLK, F), lambda i: (i, 0)),
        out_shape=jax.ShapeDtypeStruct((N, F), jnp.float32),
    )(pa, pb, hs, dinv_col, b_row)


def kernel(x, adj, W1, b1, W2, b2, W3, b3):
    src = adj[0].astype(jnp.int32)
    dst = adj[1].astype(jnp.int32)

    cnt = _deg_kernel(dst,
                      jnp.zeros((NPAD, CW), jnp.float32),
                      jnp.ones((CHUNK, CW), jnp.float32))
    cnt_col = (cnt[0, :N, 0] + cnt[1, :N, 0]).reshape(N, 1)

    hs = _mm_scale(x, W1, cnt_col)
    p = _agg_kernel(hs, src, dst)
    hs = _combine_mm(p[0], p[1], hs, cnt_col, b1.reshape(1, F), W2)
    p = _agg_kernel(hs, src, dst)
    hs = _combine_mm(p[0], p[1], hs, cnt_col, b2.reshape(1, F), W3)
    p = _agg_kernel(hs, src, dst)
    return _final(p[0], p[1], hs, cnt_col, b3.reshape(1, F))
